# Initial kernel scaffold; baseline (speedup 1.0000x reference)
import jax
import jax.numpy as jnp
from jax.experimental import pallas as pl


def kernel(x, edge_index, W, b):
    raise NotImplementedError("write your pallas kernel here")


# speedup vs baseline: 31.6612x; 31.6612x over previous
import functools

import jax
import jax.numpy as jnp
from jax import lax
from jax.experimental import pallas as pl
from jax.experimental.pallas import tpu as pltpu
from jax.experimental.pallas import tpu_sc as plsc

N = 10000
E = 320000
CH = 128
NPAD = 10240
NC = 2
NS = 16
NW = NC * NS
EPW = E // NW
K = 125
NCHUNK = EPW // K
RPT = NPAD // NS

_sc_mesh = plsc.VectorSubcoreMesh(
    core_axis_name="c", subcore_axis_name="s", num_cores=NC, num_subcores=NS
)
_sc_params = pltpu.CompilerParams(needs_layout_passes=False)


@functools.partial(
    pl.kernel,
    out_type=jax.ShapeDtypeStruct((NW, NPAD), jnp.float32),
    mesh=_sc_mesh,
    compiler_params=_sc_params,
    scratch_types=[
        pltpu.VMEM((EPW,), jnp.int32),
        pltpu.VMEM((NPAD,), jnp.float32),
    ],
)
def _deg_kernel(dst_hbm, out_hbm, idx_v, hist_v):
    wid = lax.axis_index("c") * NS + lax.axis_index("s")
    pltpu.sync_copy(dst_hbm.at[pl.ds(wid * EPW, EPW)], idx_v)

    zeros16 = jnp.zeros((16,), jnp.float32)

    def zbody(i, carry):
        hist_v[pl.ds(i * 16, 16)] = zeros16
        return carry

    lax.fori_loop(0, NPAD // 16, zbody, 0)

    ones16 = jnp.ones((16,), jnp.float32)

    def hbody(g, carry):
        idx = idx_v[pl.ds(g * 16, 16)]
        plsc.addupdate_scatter(hist_v, [idx], ones16)
        return carry

    lax.fori_loop(0, EPW // 16, hbody, 0)

    pltpu.sync_copy(hist_v, out_hbm.at[wid])


def _mm_body(x_ref, w_ref, h_ref, y_ref, dis_ref):
    deg = jnp.sum(h_ref[...], axis=0) + 1.0
    dis = lax.rsqrt(deg)
    z = jnp.dot(x_ref[...], w_ref[...], preferred_element_type=jnp.float32)
    y_ref[...] = z * dis[:, None]
    dis_ref[...] = dis[:, None]


_MM_BLK = 1024
_mm_call = pl.pallas_call(
    _mm_body,
    grid=(NPAD // _MM_BLK,),
    in_specs=[
        pl.BlockSpec((_MM_BLK, CH), lambda i: (i, 0)),
        pl.BlockSpec((CH, CH), lambda i: (0, 0)),
        pl.BlockSpec((NW, _MM_BLK), lambda i: (0, i)),
    ],
    out_specs=[
        pl.BlockSpec((_MM_BLK, CH), lambda i: (i, 0)),
        pl.BlockSpec((_MM_BLK, 1), lambda i: (i, 0)),
    ],
    out_shape=[
        jax.ShapeDtypeStruct((NPAD, CH), jnp.float32),
        jax.ShapeDtypeStruct((NPAD, 1), jnp.float32),
    ],
)


@functools.partial(
    pl.kernel,
    out_type=jax.ShapeDtypeStruct((NC, NPAD, CH), jnp.float32),
    mesh=_sc_mesh,
    compiler_params=_sc_params,
    scratch_types=[
        pltpu.VMEM((NCHUNK, K), jnp.int32),
        pltpu.VMEM((NCHUNK, K), jnp.int32),
        pltpu.VMEM((K, CH), jnp.float32),
        pltpu.VMEM_SHARED((NPAD, CH), jnp.float32),
    ],
)
def _agg_kernel(y_hbm, z_hbm, src_hbm, dst_hbm, out_hbm, src_v, dst_v, rows_v, acc):
    core = lax.axis_index("c")
    sub = lax.axis_index("s")
    wid = core * NS + sub
    sl = pl.ds(sub * RPT, RPT)

    @pl.when(core == 0)
    def _():
        pltpu.sync_copy(y_hbm.at[sl], acc.at[sl])

    @pl.when(core == 1)
    def _():
        pltpu.sync_copy(z_hbm.at[sl], acc.at[sl])

    pltpu.sync_copy(src_hbm.at[wid], src_v)
    pltpu.sync_copy(dst_hbm.at[wid], dst_v)
    plsc.subcore_barrier()

    def body(j, carry):
        pltpu.sync_copy(y_hbm.at[src_v.at[j]], rows_v)
        pltpu.sync_copy(rows_v, acc.at[dst_v.at[j]], add=True)
        return carry

    lax.fori_loop(0, NCHUNK, body, 0)

    plsc.subcore_barrier()
    pltpu.sync_copy(acc.at[sl], out_hbm.at[core].at[sl])


def _fin_body(p_ref, dis_ref, b_ref, o_ref):
    s = p_ref[0] + p_ref[1]
    o_ref[...] = s * dis_ref[...] + b_ref[...]


_FIN_BLK = 1000
_fin_call = pl.pallas_call(
    _fin_body,
    grid=(N // _FIN_BLK,),
    in_specs=[
        pl.BlockSpec((NC, _FIN_BLK, CH), lambda i: (0, i, 0)),
        pl.BlockSpec((_FIN_BLK, 1), lambda i: (i, 0)),
        pl.BlockSpec((1, CH), lambda i: (0, 0)),
    ],
    out_specs=pl.BlockSpec((_FIN_BLK, CH), lambda i: (i, 0)),
    out_shape=jax.ShapeDtypeStruct((N, CH), jnp.float32),
)


def kernel(x, edge_index, W, b):
    src = edge_index[0].astype(jnp.int32)
    dst = edge_index[1].astype(jnp.int32)
    hist = _deg_kernel(dst)
    x_pad = jnp.pad(x, ((0, NPAD - N), (0, 0)))
    yp, dis = _mm_call(x_pad, W, hist)
    zeros = jnp.zeros((NPAD, CH), jnp.float32)
    parts = _agg_kernel(
        yp, zeros, src.reshape(NW, NCHUNK, K), dst.reshape(NW, NCHUNK, K)
    )
    return _fin_call(parts, dis, b.reshape(1, CH))
